# Initial kernel scaffold; baseline (speedup 1.0000x reference)
import jax
import jax.numpy as jnp
from jax.experimental import pallas as pl


def kernel(pos, h, conditions, t, eps, We, Wt, Wcnd, Wm1, bm1, Wm2, bm2, Wn1, bn1, Wn2, bn2, Wx, Wo, bo, mask, edge_index):
    raise NotImplementedError("write your pallas kernel here")


# speedup vs baseline: 2.5507x; 2.5507x over previous
import functools

import jax
import jax.numpy as jnp
from jax import lax
from jax.experimental import pallas as pl
from jax.experimental.pallas import tpu as pltpu
from jax.experimental.pallas import tpu_sc as plsc

N = 10000
E = 320000
B = 64
POS_DIM = 3
NF = 128
HF = NF - POS_DIM
HID = 128
TT = 1000.0
G0 = -10.0
G1 = 10.0

TW = HID + 16
NC = 2
NS = 16
NW = NC * NS
EPW = E // NW
KE = 80
NBLK = EPW // KE
EBLK = 2560
NEB = E // EBLK

_f32 = jnp.float32


NBLK_N = 2000
NGB = N // NBLK_N


def _stats_body(pos_ref, eps3_ref, mask_r_ref, t_ref, stats_ref, snr_ref):
    i = pl.program_id(0)

    @pl.when(i == 0)
    def _():
        stats_ref[...] = jnp.zeros_like(stats_ref)
        t = t_ref[...]
        gt = G0 + (G1 - G0) * t
        gs = G0 + (G1 - G0) * (t - 1.0 / TT)
        snr_ref[...] = 1.0 - jnp.exp(-(gs - gt))

    msk_r = mask_r_ref[0]
    onehotT = (lax.broadcasted_iota(jnp.int32, (B, NBLK_N), 0)
               == msk_r).astype(_f32)
    possum = jnp.dot(onehotT, pos_ref[...], preferred_element_type=_f32)
    epssum = jnp.dot(onehotT, eps3_ref[...], preferred_element_type=_f32)
    cnt = jnp.sum(onehotT, axis=1, keepdims=True)
    stats_ref[...] += jnp.concatenate(
        [possum, epssum, cnt, jnp.zeros((B, 1), _f32)], axis=1)


def _stats(pos, eps3, mask_r, t):
    return pl.pallas_call(
        _stats_body,
        grid=(NGB,),
        in_specs=[
            pl.BlockSpec((NBLK_N, POS_DIM), lambda i: (i, 0)),
            pl.BlockSpec((NBLK_N, POS_DIM), lambda i: (i, 0)),
            pl.BlockSpec((1, 1, NBLK_N), lambda i: (i, 0, 0)),
            pl.BlockSpec((B, 1), lambda i: (0, 0)),
        ],
        out_specs=[pl.BlockSpec((B, 8), lambda i: (0, 0)),
                   pl.BlockSpec((B, 1), lambda i: (0, 0))],
        out_shape=[jax.ShapeDtypeStruct((B, 8), _f32),
                   jax.ShapeDtypeStruct((B, 1), _f32)],
    )(pos, eps3, mask_r, t)


def _pre_body(stats_ref, t_ref, cond_ref, pos_ref, h_ref, eps_ref, mask_c_ref,
              We_ref, Wt_ref, Wcnd_ref, Wm1_ref, bm1_ref,
              epsc_ref, h0_ref, tsrc_ref, tdst_ref):
    stats = stats_ref[...]
    cnt = jnp.maximum(stats[:, 6:7], 1.0)
    posm = stats[:, 0:3] / cnt
    epsm = stats[:, 3:6] / cnt

    msk_c = mask_c_ref[...]
    onehot = (msk_c == lax.broadcasted_iota(
        jnp.int32, (NBLK_N, B), 1)).astype(_f32)

    posc = pos_ref[...] - jnp.dot(onehot, posm, preferred_element_type=_f32)
    ep = eps_ref[...]
    ep3 = ep[:, :POS_DIM]
    eph = ep[:, POS_DIM:]
    epsc = ep3 - jnp.dot(onehot, epsm, preferred_element_type=_f32)

    t = t_ref[...]
    gt = G0 + (G1 - G0) * t
    alpha = jnp.sqrt(jax.nn.sigmoid(-gt))
    sigma = jnp.sqrt(jax.nn.sigmoid(gt))
    al = jnp.dot(onehot, alpha, preferred_element_type=_f32)
    sg = jnp.dot(onehot, sigma, preferred_element_type=_f32)

    zx = al * posc + sg * epsc
    zh = al * h_ref[...] + sg * eph

    tn = jnp.dot(onehot, t, preferred_element_type=_f32)
    cn = jnp.dot(onehot, cond_ref[...], preferred_element_type=_f32)
    h0 = (jnp.dot(zh, We_ref[...], preferred_element_type=_f32)
          + tn * Wt_ref[...] + cn * Wcnd_ref[...])

    Wm1 = Wm1_ref[...]
    A = jnp.dot(h0, Wm1[0:HID], preferred_element_type=_f32) + bm1_ref[...]
    Bt = jnp.dot(h0, Wm1[HID:2 * HID], preferred_element_type=_f32)
    xpad = jnp.concatenate(
        [zx, jnp.zeros((NBLK_N, 16 - POS_DIM), _f32)], axis=1)

    epsc_ref[...] = epsc
    h0_ref[...] = h0
    tsrc_ref[...] = jnp.concatenate([A, xpad], axis=1)
    tdst_ref[...] = jnp.concatenate([Bt, xpad], axis=1)


def _pre(stats, t, cond, pos, h, eps, mask_c, We, Wt, Wcnd, Wm1, bm1):
    full = lambda r, c: pl.BlockSpec((r, c), lambda i: (0, 0))
    return pl.pallas_call(
        _pre_body,
        grid=(NGB,),
        in_specs=[
            full(B, 8),
            full(B, 1),
            full(B, 1),
            pl.BlockSpec((NBLK_N, POS_DIM), lambda i: (i, 0)),
            pl.BlockSpec((NBLK_N, HF), lambda i: (i, 0)),
            pl.BlockSpec((NBLK_N, NF), lambda i: (i, 0)),
            pl.BlockSpec((NBLK_N, 1), lambda i: (i, 0)),
            full(HF, HID),
            full(1, HID),
            full(1, HID),
            full(2 * HID + 1, HID),
            full(1, HID),
        ],
        out_specs=[
            pl.BlockSpec((NBLK_N, POS_DIM), lambda i: (i, 0)),
            pl.BlockSpec((NBLK_N, HID), lambda i: (i, 0)),
            pl.BlockSpec((NBLK_N, TW), lambda i: (i, 0)),
            pl.BlockSpec((NBLK_N, TW), lambda i: (i, 0)),
        ],
        out_shape=[
            jax.ShapeDtypeStruct((N, POS_DIM), _f32),
            jax.ShapeDtypeStruct((N, HID), _f32),
            jax.ShapeDtypeStruct((N, TW), _f32),
            jax.ShapeDtypeStruct((N, TW), _f32),
        ],
    )(stats, t, cond, pos, h, eps, mask_c, We, Wt, Wcnd, Wm1, bm1)


def _sc_gather(tsrc, tdst, src, dst):
    mesh = plsc.VectorSubcoreMesh(core_axis_name="c", subcore_axis_name="s")

    @functools.partial(
        pl.kernel,
        out_type=[jax.ShapeDtypeStruct((E, TW), _f32),
                  jax.ShapeDtypeStruct((E, TW), _f32)],
        mesh=mesh,
        scratch_types=[
            pltpu.VMEM((KE,), jnp.int32),
            pltpu.VMEM((KE,), jnp.int32),
            pltpu.VMEM((KE, TW), _f32),
            pltpu.VMEM((KE, TW), _f32),
            pltpu.SemaphoreType.DMA,
            pltpu.SemaphoreType.DMA,
        ],
        compiler_params=pltpu.CompilerParams(use_tc_tiling_on_sc=False),
    )
    def gather_k(tsrc_hbm, tdst_hbm, src_hbm, dst_hbm, gs_hbm, gd_hbm,
                 idx_s, idx_d, rows_s, rows_d, sem_s, sem_d):
        wid = lax.axis_index("s") * NC + lax.axis_index("c")
        base0 = wid * EPW

        def body(j, _):
            base = base0 + j * KE
            pltpu.sync_copy(src_hbm.at[pl.ds(base, KE)], idx_s)
            pltpu.sync_copy(dst_hbm.at[pl.ds(base, KE)], idx_d)
            cs = pltpu.async_copy(tsrc_hbm.at[idx_s], rows_s, sem_s)
            cd = pltpu.async_copy(tdst_hbm.at[idx_d], rows_d, sem_d)
            cs.wait()
            cd.wait()
            pltpu.sync_copy(rows_s, gs_hbm.at[pl.ds(base, KE)])
            pltpu.sync_copy(rows_d, gd_hbm.at[pl.ds(base, KE)])
            return 0

        lax.fori_loop(0, NBLK, body, 0)

    return gather_k(tsrc, tdst, src, dst)


def _edge_body(gs_ref, gd_ref, Wm2_ref, bm2_ref, Wx_ref, wd2_ref, out_ref):
    gs = gs_ref[...]
    gd = gd_ref[...]
    a = gs[:, :HID]
    b = gd[:, :HID]
    rel = gs[:, HID:] - gd[:, HID:]
    d2 = jnp.sum(rel * rel, axis=1, keepdims=True)
    q = jnp.maximum(a + b + d2 * wd2_ref[...], 0.0)
    m = jnp.maximum(jnp.dot(q, Wm2_ref[...], preferred_element_type=_f32)
                    + bm2_ref[...], 0.0)
    s = jnp.tanh(jnp.dot(m, Wx_ref[...], preferred_element_type=_f32))
    out_ref[...] = jnp.concatenate([m, rel * s], axis=1)


def _edge_mlp(gs, gd, Wm2, bm2, Wx, wd2):
    return pl.pallas_call(
        _edge_body,
        grid=(NEB,),
        in_specs=[
            pl.BlockSpec((EBLK, TW), lambda i: (i, 0)),
            pl.BlockSpec((EBLK, TW), lambda i: (i, 0)),
            pl.BlockSpec((HID, HID), lambda i: (0, 0)),
            pl.BlockSpec((1, HID), lambda i: (0, 0)),
            pl.BlockSpec((HID, 1), lambda i: (0, 0)),
            pl.BlockSpec((1, HID), lambda i: (0, 0)),
        ],
        out_specs=pl.BlockSpec((EBLK, TW), lambda i: (i, 0)),
        out_shape=jax.ShapeDtypeStruct((E, TW), _f32),
    )(gs, gd, Wm2, bm2, Wx, wd2)


def _sc_scatter(rows, dst, zeros_n):
    mesh = plsc.VectorSubcoreMesh(core_axis_name="c", subcore_axis_name="s")

    @functools.partial(
        pl.kernel,
        out_type=jax.ShapeDtypeStruct((NC * N, TW), _f32),
        mesh=mesh,
        scratch_types=[
            pltpu.VMEM((KE,), jnp.int32),
            pltpu.VMEM((KE, TW), _f32),
            pltpu.MemorySpace.VMEM_SHARED((N, TW), _f32),
        ],
        compiler_params=pltpu.CompilerParams(use_tc_tiling_on_sc=False),
    )
    def scatter_k(rows_hbm, dst_hbm, zeros_hbm, out_hbm, idx_d, buf, acc):
        cid = lax.axis_index("c")
        sid = lax.axis_index("s")
        wid = sid * NC + cid
        base0 = wid * EPW

        @pl.when(sid == 0)
        def _():
            pltpu.sync_copy(zeros_hbm, acc)

        plsc.subcore_barrier()

        def body(j, _):
            base = base0 + j * KE
            pltpu.sync_copy(dst_hbm.at[pl.ds(base, KE)], idx_d)
            pltpu.sync_copy(rows_hbm.at[pl.ds(base, KE)], buf)
            pltpu.sync_copy(buf, acc.at[idx_d], add=True)
            return 0

        lax.fori_loop(0, NBLK, body, 0)
        plsc.subcore_barrier()

        @pl.when(sid == 0)
        def _():
            pltpu.sync_copy(acc, out_hbm.at[pl.ds(cid * N, N)])

    return scatter_k(rows, dst, zeros_n)


def _xstats_body(pa_ref, pb_ref, mask_r_ref, stats_ref):
    i = pl.program_id(0)

    @pl.when(i == 0)
    def _():
        stats_ref[...] = jnp.zeros_like(stats_ref)

    xagg = (pa_ref[:, HID:HID + POS_DIM]
            + pb_ref[:, HID:HID + POS_DIM])
    msk_r = mask_r_ref[0]
    onehotT = (lax.broadcasted_iota(jnp.int32, (B, NBLK_N), 0)
               == msk_r).astype(_f32)
    xsum = jnp.dot(onehotT, xagg, preferred_element_type=_f32)
    cnt = jnp.sum(onehotT, axis=1, keepdims=True)
    stats_ref[...] += jnp.concatenate(
        [xsum, cnt, jnp.zeros((B, 4), _f32)], axis=1)


def _xstats(pacc, mask_r):
    return pl.pallas_call(
        _xstats_body,
        grid=(NGB,),
        in_specs=[
            pl.BlockSpec((NBLK_N, TW), lambda i: (i, 0)),
            pl.BlockSpec((NBLK_N, TW), lambda i: (i + NGB, 0)),
            pl.BlockSpec((1, 1, NBLK_N), lambda i: (i, 0, 0)),
        ],
        out_specs=pl.BlockSpec((B, 8), lambda i: (0, 0)),
        out_shape=jax.ShapeDtypeStruct((B, 8), _f32),
    )(pacc, pacc, mask_r)


def _post_body(stats_ref, pa_ref, pb_ref, h0_ref, Wn1_ref, bn1_ref, Wn2_ref,
               bn2_ref, Wo_ref, bo_ref, mask_c_ref, mask_r_ref, epsc_ref,
               epsh_ref, err_ref, epos_ref, eh_ref):
    i = pl.program_id(0)

    @pl.when(i == 0)
    def _():
        err_ref[...] = jnp.zeros_like(err_ref)

    acc = pa_ref[...] + pb_ref[...]
    agg = acc[:, :HID]
    xagg = acc[:, HID:HID + POS_DIM]

    h0 = h0_ref[...]
    Wn1 = Wn1_ref[...]
    pre = (jnp.dot(h0, Wn1[0:HID], preferred_element_type=_f32)
           + jnp.dot(agg, Wn1[HID:2 * HID], preferred_element_type=_f32)
           + bn1_ref[...])
    h1 = h0 + jnp.dot(jnp.maximum(pre, 0.0), Wn2_ref[...],
                      preferred_element_type=_f32) + bn2_ref[...]

    stats = stats_ref[...]
    cnt = jnp.maximum(stats[:, 3:4], 1.0)
    xm = stats[:, 0:3] / cnt

    msk_c = mask_c_ref[...]
    onehot = (msk_c == lax.broadcasted_iota(
        jnp.int32, (NBLK_N, B), 1)).astype(_f32)
    msk_r = mask_r_ref[0]
    onehotT = (lax.broadcasted_iota(jnp.int32, (B, NBLK_N), 0)
               == msk_r).astype(_f32)

    eps_pos = xagg - jnp.dot(onehot, xm, preferred_element_type=_f32)
    eps_h = jnp.dot(h1, Wo_ref[...], preferred_element_type=_f32) + bo_ref[...]

    perr = epsc_ref[...] - eps_pos
    herr = epsh_ref[...] - eps_h
    err = (jnp.sum(perr * perr, axis=1, keepdims=True)
           + jnp.sum(herr * herr, axis=1, keepdims=True))
    err_ref[...] += jnp.dot(onehotT, err, preferred_element_type=_f32)
    epos_ref[...] = eps_pos
    eh_ref[...] = eps_h


def _post(stats, pacc, h0, Wn1, bn1, Wn2, bn2, Wo, bo, mask_c, mask_r,
          epsc, epsh):
    full = lambda r, c: pl.BlockSpec((r, c), lambda i: (0, 0))
    return pl.pallas_call(
        _post_body,
        grid=(NGB,),
        in_specs=[
            full(B, 8),
            pl.BlockSpec((NBLK_N, TW), lambda i: (i, 0)),
            pl.BlockSpec((NBLK_N, TW), lambda i: (i + NGB, 0)),
            pl.BlockSpec((NBLK_N, HID), lambda i: (i, 0)),
            full(2 * HID, HID),
            full(1, HID),
            full(HID, HID),
            full(1, HID),
            full(HID, HF),
            full(1, HF),
            pl.BlockSpec((NBLK_N, 1), lambda i: (i, 0)),
            pl.BlockSpec((1, 1, NBLK_N), lambda i: (i, 0, 0)),
            pl.BlockSpec((NBLK_N, POS_DIM), lambda i: (i, 0)),
            pl.BlockSpec((NBLK_N, HF), lambda i: (i, 0)),
        ],
        out_specs=[
            pl.BlockSpec((B, 1), lambda i: (0, 0)),
            pl.BlockSpec((NBLK_N, POS_DIM), lambda i: (i, 0)),
            pl.BlockSpec((NBLK_N, HF), lambda i: (i, 0)),
        ],
        out_shape=[
            jax.ShapeDtypeStruct((B, 1), _f32),
            jax.ShapeDtypeStruct((N, POS_DIM), _f32),
            jax.ShapeDtypeStruct((N, HF), _f32),
        ],
    )(stats, pacc, pacc, h0, Wn1, bn1, Wn2, bn2, Wo, bo, mask_c, mask_r,
      epsc, epsh)


def kernel(pos, h, conditions, t, eps, We, Wt, Wcnd, Wm1, bm1, Wm2, bm2,
           Wn1, bn1, Wn2, bn2, Wx, Wo, bo, mask, edge_index):
    mask = mask.astype(jnp.int32)
    mask_c = mask.reshape(N, 1)
    mask_r = mask.reshape(NGB, 1, NBLK_N)
    src = edge_index[0].astype(jnp.int32)
    dst = edge_index[1].astype(jnp.int32)

    stats, snr = _stats(pos, eps[:, :POS_DIM], mask_r, t)
    epsc, h0, tsrc, tdst = _pre(
        stats, t, conditions, pos, h, eps, mask_c,
        We, Wt.reshape(1, HID), Wcnd.reshape(1, HID), Wm1, bm1.reshape(1, HID))

    gs, gd = _sc_gather(tsrc, tdst, src, dst)
    rows = _edge_mlp(gs, gd, Wm2, bm2.reshape(1, HID), Wx,
                     Wm1[2 * HID].reshape(1, HID))
    pacc = _sc_scatter(rows, dst, jnp.zeros((N, TW), _f32))

    xstats = _xstats(pacc, mask_r)
    error_t, eps_pos, eps_h = _post(
        xstats, pacc, h0, Wn1, bn1.reshape(1, HID), Wn2, bn2.reshape(1, HID),
        Wo, bo.reshape(1, HF), mask_c, mask_r, epsc, eps[:, POS_DIM:])

    net_eps_xh = jnp.concatenate([eps_pos, eps_h], axis=1)
    eps_xh = jnp.concatenate([epsc, eps[:, POS_DIM:]], axis=1)
    return (error_t.reshape(B), snr.reshape(B), net_eps_xh, eps_xh)


# speedup vs baseline: 3.8683x; 1.5165x over previous
import functools

import jax
import jax.numpy as jnp
from jax import lax
from jax.experimental import pallas as pl
from jax.experimental.pallas import tpu as pltpu
from jax.experimental.pallas import tpu_sc as plsc

N = 10000
E = 320000
B = 64
POS_DIM = 3
NF = 128
HF = NF - POS_DIM
HID = 128
TT = 1000.0
G0 = -10.0
G1 = 10.0

NC = 2
NS = 16
NW = NC * NS
EPW = E // NW
KE = 80
NBLK = EPW // KE
EBLK = 2560
NEB = E // EBLK
CW = 8

_f32 = jnp.float32


NBLK_N = 2000
NGB = N // NBLK_N


def _stats_body(pos_ref, eps3_ref, mask_r_ref, t_ref, stats_ref, snr_ref):
    i = pl.program_id(0)

    @pl.when(i == 0)
    def _():
        stats_ref[...] = jnp.zeros_like(stats_ref)
        t = t_ref[...]
        gt = G0 + (G1 - G0) * t
        gs = G0 + (G1 - G0) * (t - 1.0 / TT)
        snr_ref[...] = 1.0 - jnp.exp(-(gs - gt))

    msk_r = mask_r_ref[0]
    onehotT = (lax.broadcasted_iota(jnp.int32, (B, NBLK_N), 0)
               == msk_r).astype(_f32)
    possum = jnp.dot(onehotT, pos_ref[...], preferred_element_type=_f32)
    epssum = jnp.dot(onehotT, eps3_ref[...], preferred_element_type=_f32)
    cnt = jnp.sum(onehotT, axis=1, keepdims=True)
    stats_ref[...] += jnp.concatenate(
        [possum, epssum, cnt, jnp.zeros((B, 1), _f32)], axis=1)


def _stats(pos, eps3, mask_r, t):
    return pl.pallas_call(
        _stats_body,
        grid=(NGB,),
        in_specs=[
            pl.BlockSpec((NBLK_N, POS_DIM), lambda i: (i, 0)),
            pl.BlockSpec((NBLK_N, POS_DIM), lambda i: (i, 0)),
            pl.BlockSpec((1, 1, NBLK_N), lambda i: (i, 0, 0)),
            pl.BlockSpec((B, 1), lambda i: (0, 0)),
        ],
        out_specs=[pl.BlockSpec((B, 8), lambda i: (0, 0)),
                   pl.BlockSpec((B, 1), lambda i: (0, 0))],
        out_shape=[jax.ShapeDtypeStruct((B, 8), _f32),
                   jax.ShapeDtypeStruct((B, 1), _f32)],
    )(pos, eps3, mask_r, t)


def _pre_body(stats_ref, t_ref, cond_ref, pos_ref, h_ref, eps_ref, mask_c_ref,
              We_ref, Wt_ref, Wcnd_ref, Wm1_ref, bm1_ref,
              epsc_ref, h0_ref, tsrc_ref, tdst_ref, zx_ref):
    stats = stats_ref[...]
    cnt = jnp.maximum(stats[:, 6:7], 1.0)
    posm = stats[:, 0:3] / cnt
    epsm = stats[:, 3:6] / cnt

    msk_c = mask_c_ref[...]
    onehot = (msk_c == lax.broadcasted_iota(
        jnp.int32, (NBLK_N, B), 1)).astype(_f32)

    posc = pos_ref[...] - jnp.dot(onehot, posm, preferred_element_type=_f32)
    ep = eps_ref[...]
    ep3 = ep[:, :POS_DIM]
    eph = ep[:, POS_DIM:]
    epsc = ep3 - jnp.dot(onehot, epsm, preferred_element_type=_f32)

    t = t_ref[...]
    gt = G0 + (G1 - G0) * t
    alpha = jnp.sqrt(jax.nn.sigmoid(-gt))
    sigma = jnp.sqrt(jax.nn.sigmoid(gt))
    al = jnp.dot(onehot, alpha, preferred_element_type=_f32)
    sg = jnp.dot(onehot, sigma, preferred_element_type=_f32)

    zx = al * posc + sg * epsc
    zh = al * h_ref[...] + sg * eph

    tn = jnp.dot(onehot, t, preferred_element_type=_f32)
    cn = jnp.dot(onehot, cond_ref[...], preferred_element_type=_f32)
    h0 = (jnp.dot(zh, We_ref[...], preferred_element_type=_f32)
          + tn * Wt_ref[...] + cn * Wcnd_ref[...])

    Wm1 = Wm1_ref[...]
    A = jnp.dot(h0, Wm1[0:HID], preferred_element_type=_f32) + bm1_ref[...]
    Bt = jnp.dot(h0, Wm1[HID:2 * HID], preferred_element_type=_f32)

    epsc_ref[...] = epsc
    h0_ref[...] = h0
    tsrc_ref[...] = A
    tdst_ref[...] = Bt
    zx_ref[...] = zx


def _pre(stats, t, cond, pos, h, eps, mask_c, We, Wt, Wcnd, Wm1, bm1):
    full = lambda r, c: pl.BlockSpec((r, c), lambda i: (0, 0))
    return pl.pallas_call(
        _pre_body,
        grid=(NGB,),
        in_specs=[
            full(B, 8),
            full(B, 1),
            full(B, 1),
            pl.BlockSpec((NBLK_N, POS_DIM), lambda i: (i, 0)),
            pl.BlockSpec((NBLK_N, HF), lambda i: (i, 0)),
            pl.BlockSpec((NBLK_N, NF), lambda i: (i, 0)),
            pl.BlockSpec((NBLK_N, 1), lambda i: (i, 0)),
            full(HF, HID),
            full(1, HID),
            full(1, HID),
            full(2 * HID + 1, HID),
            full(1, HID),
        ],
        out_specs=[
            pl.BlockSpec((NBLK_N, POS_DIM), lambda i: (i, 0)),
            pl.BlockSpec((NBLK_N, HID), lambda i: (i, 0)),
            pl.BlockSpec((NBLK_N, HID), lambda i: (i, 0)),
            pl.BlockSpec((NBLK_N, HID), lambda i: (i, 0)),
            pl.BlockSpec((NBLK_N, POS_DIM), lambda i: (i, 0)),
        ],
        out_shape=[
            jax.ShapeDtypeStruct((N, POS_DIM), _f32),
            jax.ShapeDtypeStruct((N, HID), _f32),
            jax.ShapeDtypeStruct((N, HID), _f32),
            jax.ShapeDtypeStruct((N, HID), _f32),
            jax.ShapeDtypeStruct((N, POS_DIM), _f32),
        ],
    )(stats, t, cond, pos, h, eps, mask_c, We, Wt, Wcnd, Wm1, bm1)


def _sc_gather(tsrc, tdst, src, dst):
    mesh = plsc.VectorSubcoreMesh(core_axis_name="c", subcore_axis_name="s")

    @functools.partial(
        pl.kernel,
        out_type=[jax.ShapeDtypeStruct((E, HID), _f32),
                  jax.ShapeDtypeStruct((E, HID), _f32)],
        mesh=mesh,
        scratch_types=[
            pltpu.VMEM((KE,), jnp.int32),
            pltpu.VMEM((KE,), jnp.int32),
            pltpu.VMEM((KE, HID), _f32),
            pltpu.VMEM((KE, HID), _f32),
            pltpu.SemaphoreType.DMA,
            pltpu.SemaphoreType.DMA,
        ],
        compiler_params=pltpu.CompilerParams(use_tc_tiling_on_sc=True),
    )
    def gather_k(tsrc_hbm, tdst_hbm, src_hbm, dst_hbm, gs_hbm, gd_hbm,
                 idx_s, idx_d, rows_s, rows_d, sem_s, sem_d):
        wid = lax.axis_index("s") * NC + lax.axis_index("c")
        base0 = wid * EPW

        def body(j, _):
            base = base0 + j * KE
            pltpu.sync_copy(src_hbm.at[pl.ds(base, KE)], idx_s)
            pltpu.sync_copy(dst_hbm.at[pl.ds(base, KE)], idx_d)
            cs = pltpu.async_copy(tsrc_hbm.at[idx_s], rows_s, sem_s)
            cd = pltpu.async_copy(tdst_hbm.at[idx_d], rows_d, sem_d)
            cs.wait()
            cd.wait()
            pltpu.sync_copy(rows_s, gs_hbm.at[pl.ds(base, KE)])
            pltpu.sync_copy(rows_d, gd_hbm.at[pl.ds(base, KE)])
            return 0

        lax.fori_loop(0, NBLK, body, 0)

    return gather_k(tsrc, tdst, src, dst)


def _sc_rel(x0, x1, x2, src, dst):
    mesh = plsc.VectorSubcoreMesh(core_axis_name="c", subcore_axis_name="s")

    @functools.partial(
        pl.kernel,
        out_type=[jax.ShapeDtypeStruct((E,), _f32),
                  jax.ShapeDtypeStruct((E,), _f32),
                  jax.ShapeDtypeStruct((E,), _f32),
                  jax.ShapeDtypeStruct((E,), _f32)],
        mesh=mesh,
        scratch_types=[
            pltpu.VMEM((KE,), jnp.int32),
            pltpu.VMEM((KE,), jnp.int32),
            pltpu.VMEM((N,), _f32),
            pltpu.VMEM((N,), _f32),
            pltpu.VMEM((N,), _f32),
            pltpu.VMEM((KE,), _f32),
            pltpu.VMEM((KE,), _f32),
            pltpu.VMEM((KE,), _f32),
            pltpu.VMEM((KE,), _f32),
        ],
        compiler_params=pltpu.CompilerParams(use_tc_tiling_on_sc=False,
                                             needs_layout_passes=False),
    )
    def rel_k(x0_hbm, x1_hbm, x2_hbm, src_hbm, dst_hbm,
              rx_hbm, ry_hbm, rz_hbm, d2_hbm,
              idx_s, idx_d, xv0, xv1, xv2, rbx, rby, rbz, rb2):
        wid = lax.axis_index("s") * NC + lax.axis_index("c")
        base0 = wid * EPW
        pltpu.sync_copy(x0_hbm, xv0)
        pltpu.sync_copy(x1_hbm, xv1)
        pltpu.sync_copy(x2_hbm, xv2)

        def body(j, _):
            base = base0 + j * KE
            pltpu.sync_copy(src_hbm.at[pl.ds(base, KE)], idx_s)
            pltpu.sync_copy(dst_hbm.at[pl.ds(base, KE)], idx_d)
            for g in range(KE // 16):
                sl = pl.ds(g * 16, 16)
                iv_s = idx_s[sl]
                iv_d = idx_d[sl]
                rx = (plsc.load_gather(xv0, [iv_s])
                      - plsc.load_gather(xv0, [iv_d]))
                ry = (plsc.load_gather(xv1, [iv_s])
                      - plsc.load_gather(xv1, [iv_d]))
                rz = (plsc.load_gather(xv2, [iv_s])
                      - plsc.load_gather(xv2, [iv_d]))
                rbx[sl] = rx
                rby[sl] = ry
                rbz[sl] = rz
                rb2[sl] = rx * rx + ry * ry + rz * rz
            pltpu.sync_copy(rbx, rx_hbm.at[pl.ds(base, KE)])
            pltpu.sync_copy(rby, ry_hbm.at[pl.ds(base, KE)])
            pltpu.sync_copy(rbz, rz_hbm.at[pl.ds(base, KE)])
            pltpu.sync_copy(rb2, d2_hbm.at[pl.ds(base, KE)])
            return 0

        lax.fori_loop(0, NBLK, body, 0)

    return rel_k(x0, x1, x2, src, dst)


def _edge_body(gs_ref, gd_ref, rx_ref, ry_ref, rz_ref, d2_ref,
               Wm2_ref, bm2_ref, wx_ref, wd2_ref,
               m_ref, cx_ref, cy_ref, cz_ref):
    d2r = d2_ref[0]
    outer = lax.dot_general(d2r, wd2_ref[...],
                            (((0,), (0,)), ((), ())),
                            preferred_element_type=_f32)
    q = jnp.maximum(gs_ref[...] + gd_ref[...] + outer, 0.0)
    m = jnp.maximum(jnp.dot(q, Wm2_ref[...], preferred_element_type=_f32)
                    + bm2_ref[...], 0.0)
    srow = jnp.tanh(lax.dot_general(wx_ref[...], m,
                                    (((1,), (1,)), ((), ())),
                                    preferred_element_type=_f32))
    m_ref[...] = m
    cx_ref[0] = rx_ref[0] * srow
    cy_ref[0] = ry_ref[0] * srow
    cz_ref[0] = rz_ref[0] * srow


def _edge_mlp(gs, gd, rx, ry, rz, d2, Wm2, bm2, wx, wd2):
    row = pl.BlockSpec((1, 1, EBLK), lambda i: (i, 0, 0))
    return pl.pallas_call(
        _edge_body,
        grid=(NEB,),
        in_specs=[
            pl.BlockSpec((EBLK, HID), lambda i: (i, 0)),
            pl.BlockSpec((EBLK, HID), lambda i: (i, 0)),
            row, row, row, row,
            pl.BlockSpec((HID, HID), lambda i: (0, 0)),
            pl.BlockSpec((1, HID), lambda i: (0, 0)),
            pl.BlockSpec((1, HID), lambda i: (0, 0)),
            pl.BlockSpec((1, HID), lambda i: (0, 0)),
        ],
        out_specs=[pl.BlockSpec((EBLK, HID), lambda i: (i, 0)),
                   row, row, row],
        out_shape=[jax.ShapeDtypeStruct((E, HID), _f32),
                   jax.ShapeDtypeStruct((NEB, 1, EBLK), _f32),
                   jax.ShapeDtypeStruct((NEB, 1, EBLK), _f32),
                   jax.ShapeDtypeStruct((NEB, 1, EBLK), _f32)],
    )(gs, gd, rx, ry, rz, d2, Wm2, bm2, wx, wd2)


def _sc_scatter(rows, dst, zeros_n):
    mesh = plsc.VectorSubcoreMesh(core_axis_name="c", subcore_axis_name="s")

    @functools.partial(
        pl.kernel,
        out_type=jax.ShapeDtypeStruct((NC * N, HID), _f32),
        mesh=mesh,
        scratch_types=[
            pltpu.VMEM((KE,), jnp.int32),
            pltpu.VMEM((KE, HID), _f32),
            pltpu.MemorySpace.VMEM_SHARED((N, HID), _f32),
        ],
        compiler_params=pltpu.CompilerParams(use_tc_tiling_on_sc=True),
    )
    def scatter_k(rows_hbm, dst_hbm, zeros_hbm, out_hbm, idx_d, buf, acc):
        cid = lax.axis_index("c")
        sid = lax.axis_index("s")
        wid = sid * NC + cid
        base0 = wid * EPW

        @pl.when(sid == 0)
        def _():
            pltpu.sync_copy(zeros_hbm, acc)

        plsc.subcore_barrier()

        def body(j, _):
            base = base0 + j * KE
            pltpu.sync_copy(dst_hbm.at[pl.ds(base, KE)], idx_d)
            pltpu.sync_copy(rows_hbm.at[pl.ds(base, KE)], buf)
            pltpu.sync_copy(buf, acc.at[idx_d], add=True)
            return 0

        lax.fori_loop(0, NBLK, body, 0)
        plsc.subcore_barrier()

        @pl.when(sid == 0)
        def _():
            pltpu.sync_copy(acc, out_hbm.at[pl.ds(cid * N, N)])

    return scatter_k(rows, dst, zeros_n)


def _sc_cscatter(cx, cy, cz, dst, zeros_c):
    mesh = plsc.VectorSubcoreMesh(core_axis_name="c", subcore_axis_name="s")

    @functools.partial(
        pl.kernel,
        out_type=jax.ShapeDtypeStruct((NC * N, CW), _f32),
        mesh=mesh,
        scratch_types=[
            pltpu.VMEM((KE,), jnp.int32),
            pltpu.VMEM((KE,), _f32),
            pltpu.VMEM((KE,), _f32),
            pltpu.VMEM((KE,), _f32),
            pltpu.VMEM((KE, CW), _f32),
            pltpu.MemorySpace.VMEM_SHARED((N, CW), _f32),
        ],
        compiler_params=pltpu.CompilerParams(use_tc_tiling_on_sc=False,
                                             needs_layout_passes=False),
    )
    def cscatter_k(cx_hbm, cy_hbm, cz_hbm, dst_hbm, zeros_hbm, out_hbm,
                   idx_d, cbx, cby, cbz, rows, acc):
        cid = lax.axis_index("c")
        sid = lax.axis_index("s")
        wid = sid * NC + cid
        base0 = wid * EPW

        @pl.when(sid == 0)
        def _():
            pltpu.sync_copy(zeros_hbm, acc)

        zv = jnp.zeros((16,), _f32)
        for g in range(KE // 16):
            for c in range(CW):
                plsc.store_scatter(
                    rows,
                    [lax.iota(jnp.int32, 16) + (g * 16),
                     jnp.full((16,), c, jnp.int32)], zv)

        plsc.subcore_barrier()

        def body(j, _):
            base = base0 + j * KE
            pltpu.sync_copy(dst_hbm.at[pl.ds(base, KE)], idx_d)
            pltpu.sync_copy(cx_hbm.at[pl.ds(base, KE)], cbx)
            pltpu.sync_copy(cy_hbm.at[pl.ds(base, KE)], cby)
            pltpu.sync_copy(cz_hbm.at[pl.ds(base, KE)], cbz)
            for g in range(KE // 16):
                sl = pl.ds(g * 16, 16)
                rowi = lax.iota(jnp.int32, 16) + (g * 16)
                plsc.store_scatter(
                    rows, [rowi, jnp.zeros((16,), jnp.int32)], cbx[sl])
                plsc.store_scatter(
                    rows, [rowi, jnp.full((16,), 1, jnp.int32)], cby[sl])
                plsc.store_scatter(
                    rows, [rowi, jnp.full((16,), 2, jnp.int32)], cbz[sl])
            pltpu.sync_copy(rows, acc.at[idx_d], add=True)
            return 0

        lax.fori_loop(0, NBLK, body, 0)
        plsc.subcore_barrier()

        @pl.when(sid == 0)
        def _():
            pltpu.sync_copy(acc, out_hbm.at[pl.ds(cid * N, N)])

    return cscatter_k(cx, cy, cz, dst, zeros_c)


def _xstats_body(pa_ref, pb_ref, mask_r_ref, stats_ref):
    i = pl.program_id(0)

    @pl.when(i == 0)
    def _():
        stats_ref[...] = jnp.zeros_like(stats_ref)

    xagg = (pa_ref[:, :POS_DIM] + pb_ref[:, :POS_DIM])
    msk_r = mask_r_ref[0]
    onehotT = (lax.broadcasted_iota(jnp.int32, (B, NBLK_N), 0)
               == msk_r).astype(_f32)
    xsum = jnp.dot(onehotT, xagg, preferred_element_type=_f32)
    cnt = jnp.sum(onehotT, axis=1, keepdims=True)
    stats_ref[...] += jnp.concatenate(
        [xsum, cnt, jnp.zeros((B, 4), _f32)], axis=1)


def _xstats(px, mask_r):
    return pl.pallas_call(
        _xstats_body,
        grid=(NGB,),
        in_specs=[
            pl.BlockSpec((NBLK_N, CW), lambda i: (i, 0)),
            pl.BlockSpec((NBLK_N, CW), lambda i: (i + NGB, 0)),
            pl.BlockSpec((1, 1, NBLK_N), lambda i: (i, 0, 0)),
        ],
        out_specs=pl.BlockSpec((B, 8), lambda i: (0, 0)),
        out_shape=jax.ShapeDtypeStruct((B, 8), _f32),
    )(px, px, mask_r)


def _post_body(stats_ref, pa_ref, pb_ref, pxa_ref, pxb_ref, h0_ref, Wn1_ref,
               bn1_ref, Wn2_ref, bn2_ref, Wo_ref, bo_ref, mask_c_ref,
               mask_r_ref, epsc_ref, epsh_ref, err_ref, epos_ref, eh_ref):
    i = pl.program_id(0)

    @pl.when(i == 0)
    def _():
        err_ref[...] = jnp.zeros_like(err_ref)

    agg = pa_ref[...] + pb_ref[...]
    xagg = pxa_ref[:, :POS_DIM] + pxb_ref[:, :POS_DIM]

    h0 = h0_ref[...]
    Wn1 = Wn1_ref[...]
    pre = (jnp.dot(h0, Wn1[0:HID], preferred_element_type=_f32)
           + jnp.dot(agg, Wn1[HID:2 * HID], preferred_element_type=_f32)
           + bn1_ref[...])
    h1 = h0 + jnp.dot(jnp.maximum(pre, 0.0), Wn2_ref[...],
                      preferred_element_type=_f32) + bn2_ref[...]

    stats = stats_ref[...]
    cnt = jnp.maximum(stats[:, 3:4], 1.0)
    xm = stats[:, 0:3] / cnt

    msk_c = mask_c_ref[...]
    onehot = (msk_c == lax.broadcasted_iota(
        jnp.int32, (NBLK_N, B), 1)).astype(_f32)
    msk_r = mask_r_ref[0]
    onehotT = (lax.broadcasted_iota(jnp.int32, (B, NBLK_N), 0)
               == msk_r).astype(_f32)

    eps_pos = xagg - jnp.dot(onehot, xm, preferred_element_type=_f32)
    eps_h = jnp.dot(h1, Wo_ref[...], preferred_element_type=_f32) + bo_ref[...]

    perr = epsc_ref[...] - eps_pos
    herr = epsh_ref[...] - eps_h
    err = (jnp.sum(perr * perr, axis=1, keepdims=True)
           + jnp.sum(herr * herr, axis=1, keepdims=True))
    err_ref[...] += jnp.dot(onehotT, err, preferred_element_type=_f32)
    epos_ref[...] = eps_pos
    eh_ref[...] = eps_h


def _post(stats, pm, px, h0, Wn1, bn1, Wn2, bn2, Wo, bo, mask_c, mask_r,
          epsc, epsh):
    full = lambda r, c: pl.BlockSpec((r, c), lambda i: (0, 0))
    return pl.pallas_call(
        _post_body,
        grid=(NGB,),
        in_specs=[
            full(B, 8),
            pl.BlockSpec((NBLK_N, HID), lambda i: (i, 0)),
            pl.BlockSpec((NBLK_N, HID), lambda i: (i + NGB, 0)),
            pl.BlockSpec((NBLK_N, CW), lambda i: (i, 0)),
            pl.BlockSpec((NBLK_N, CW), lambda i: (i + NGB, 0)),
            pl.BlockSpec((NBLK_N, HID), lambda i: (i, 0)),
            full(2 * HID, HID),
            full(1, HID),
            full(HID, HID),
            full(1, HID),
            full(HID, HF),
            full(1, HF),
            pl.BlockSpec((NBLK_N, 1), lambda i: (i, 0)),
            pl.BlockSpec((1, 1, NBLK_N), lambda i: (i, 0, 0)),
            pl.BlockSpec((NBLK_N, POS_DIM), lambda i: (i, 0)),
            pl.BlockSpec((NBLK_N, HF), lambda i: (i, 0)),
        ],
        out_specs=[
            pl.BlockSpec((B, 1), lambda i: (0, 0)),
            pl.BlockSpec((NBLK_N, POS_DIM), lambda i: (i, 0)),
            pl.BlockSpec((NBLK_N, HF), lambda i: (i, 0)),
        ],
        out_shape=[
            jax.ShapeDtypeStruct((B, 1), _f32),
            jax.ShapeDtypeStruct((N, POS_DIM), _f32),
            jax.ShapeDtypeStruct((N, HF), _f32),
        ],
    )(stats, pm, pm, px, px, h0, Wn1, bn1, Wn2, bn2, Wo, bo, mask_c, mask_r,
      epsc, epsh)


def kernel(pos, h, conditions, t, eps, We, Wt, Wcnd, Wm1, bm1, Wm2, bm2,
           Wn1, bn1, Wn2, bn2, Wx, Wo, bo, mask, edge_index):
    mask = mask.astype(jnp.int32)
    mask_c = mask.reshape(N, 1)
    mask_r = mask.reshape(NGB, 1, NBLK_N)
    src = edge_index[0].astype(jnp.int32)
    dst = edge_index[1].astype(jnp.int32)

    stats, snr = _stats(pos, eps[:, :POS_DIM], mask_r, t)
    epsc, h0, tsrc, tdst, zx = _pre(
        stats, t, conditions, pos, h, eps, mask_c,
        We, Wt.reshape(1, HID), Wcnd.reshape(1, HID), Wm1, bm1.reshape(1, HID))

    gs, gd = _sc_gather(tsrc, tdst, src, dst)
    relx, rely, relz, d2 = _sc_rel(zx[:, 0], zx[:, 1], zx[:, 2], src, dst)
    mrows, cx, cy, cz = _edge_mlp(
        gs, gd,
        relx.reshape(NEB, 1, EBLK), rely.reshape(NEB, 1, EBLK),
        relz.reshape(NEB, 1, EBLK), d2.reshape(NEB, 1, EBLK),
        Wm2, bm2.reshape(1, HID), Wx.reshape(1, HID),
        Wm1[2 * HID].reshape(1, HID))
    pm = _sc_scatter(mrows, dst, jnp.zeros((N, HID), _f32))
    px = _sc_cscatter(cx.reshape(E), cy.reshape(E), cz.reshape(E), dst,
                      jnp.zeros((N, CW), _f32))

    xstats = _xstats(px, mask_r)
    error_t, eps_pos, eps_h = _post(
        xstats, pm, px, h0, Wn1, bn1.reshape(1, HID), Wn2, bn2.reshape(1, HID),
        Wo, bo.reshape(1, HF), mask_c, mask_r, epsc, eps[:, POS_DIM:])

    net_eps_xh = jnp.concatenate([eps_pos, eps_h], axis=1)
    eps_xh = jnp.concatenate([epsc, eps[:, POS_DIM:]], axis=1)
    return (error_t.reshape(B), snr.reshape(B), net_eps_xh, eps_xh)


# speedup vs baseline: 4.2073x; 1.0876x over previous
import functools

import jax
import jax.numpy as jnp
from jax import lax
from jax.experimental import pallas as pl
from jax.experimental.pallas import tpu as pltpu
from jax.experimental.pallas import tpu_sc as plsc

N = 10000
E = 320000
B = 64
POS_DIM = 3
NF = 128
HF = NF - POS_DIM
HID = 128
TT = 1000.0
G0 = -10.0
G1 = 10.0

NC = 2
NS = 16
NW = NC * NS
EPW = E // NW
KE = 80
NBLK = EPW // KE
EBLK = 2560
NEB = E // EBLK
CW = 8

_f32 = jnp.float32


NBLK_N = 2000
NGB = N // NBLK_N


def _stats_body(pos_ref, eps3_ref, mask_r_ref, t_ref, stats_ref, snr_ref):
    i = pl.program_id(0)

    @pl.when(i == 0)
    def _():
        stats_ref[...] = jnp.zeros_like(stats_ref)
        t = t_ref[...]
        gt = G0 + (G1 - G0) * t
        gs = G0 + (G1 - G0) * (t - 1.0 / TT)
        snr_ref[...] = 1.0 - jnp.exp(-(gs - gt))

    msk_r = mask_r_ref[0]
    onehotT = (lax.broadcasted_iota(jnp.int32, (B, NBLK_N), 0)
               == msk_r).astype(_f32)
    possum = jnp.dot(onehotT, pos_ref[...], preferred_element_type=_f32)
    epssum = jnp.dot(onehotT, eps3_ref[...], preferred_element_type=_f32)
    cnt = jnp.sum(onehotT, axis=1, keepdims=True)
    stats_ref[...] += jnp.concatenate(
        [possum, epssum, cnt, jnp.zeros((B, 1), _f32)], axis=1)


def _stats(pos, eps3, mask_r, t):
    return pl.pallas_call(
        _stats_body,
        grid=(NGB,),
        in_specs=[
            pl.BlockSpec((NBLK_N, POS_DIM), lambda i: (i, 0)),
            pl.BlockSpec((NBLK_N, POS_DIM), lambda i: (i, 0)),
            pl.BlockSpec((1, 1, NBLK_N), lambda i: (i, 0, 0)),
            pl.BlockSpec((B, 1), lambda i: (0, 0)),
        ],
        out_specs=[pl.BlockSpec((B, 8), lambda i: (0, 0)),
                   pl.BlockSpec((B, 1), lambda i: (0, 0))],
        out_shape=[jax.ShapeDtypeStruct((B, 8), _f32),
                   jax.ShapeDtypeStruct((B, 1), _f32)],
    )(pos, eps3, mask_r, t)


def _pre_body(stats_ref, t_ref, cond_ref, pos_ref, h_ref, eps_ref, mask_c_ref,
              We_ref, Wt_ref, Wcnd_ref, Wm1_ref, bm1_ref,
              epsc_ref, h0_ref, tsrc_ref, tdst_ref, zx_ref):
    stats = stats_ref[...]
    cnt = jnp.maximum(stats[:, 6:7], 1.0)
    posm = stats[:, 0:3] / cnt
    epsm = stats[:, 3:6] / cnt

    msk_c = mask_c_ref[...]
    onehot = (msk_c == lax.broadcasted_iota(
        jnp.int32, (NBLK_N, B), 1)).astype(_f32)

    posc = pos_ref[...] - jnp.dot(onehot, posm, preferred_element_type=_f32)
    ep = eps_ref[...]
    ep3 = ep[:, :POS_DIM]
    eph = ep[:, POS_DIM:]
    epsc = ep3 - jnp.dot(onehot, epsm, preferred_element_type=_f32)

    t = t_ref[...]
    gt = G0 + (G1 - G0) * t
    alpha = jnp.sqrt(jax.nn.sigmoid(-gt))
    sigma = jnp.sqrt(jax.nn.sigmoid(gt))
    al = jnp.dot(onehot, alpha, preferred_element_type=_f32)
    sg = jnp.dot(onehot, sigma, preferred_element_type=_f32)

    zx = al * posc + sg * epsc
    zh = al * h_ref[...] + sg * eph

    tn = jnp.dot(onehot, t, preferred_element_type=_f32)
    cn = jnp.dot(onehot, cond_ref[...], preferred_element_type=_f32)
    h0 = (jnp.dot(zh, We_ref[...], preferred_element_type=_f32)
          + tn * Wt_ref[...] + cn * Wcnd_ref[...])

    Wm1 = Wm1_ref[...]
    A = jnp.dot(h0, Wm1[0:HID], preferred_element_type=_f32) + bm1_ref[...]
    Bt = jnp.dot(h0, Wm1[HID:2 * HID], preferred_element_type=_f32)

    epsc_ref[...] = epsc
    h0_ref[...] = h0
    tsrc_ref[...] = A
    tdst_ref[...] = Bt
    zx_ref[...] = zx


def _pre(stats, t, cond, pos, h, eps, mask_c, We, Wt, Wcnd, Wm1, bm1):
    full = lambda r, c: pl.BlockSpec((r, c), lambda i: (0, 0))
    return pl.pallas_call(
        _pre_body,
        grid=(NGB,),
        in_specs=[
            full(B, 8),
            full(B, 1),
            full(B, 1),
            pl.BlockSpec((NBLK_N, POS_DIM), lambda i: (i, 0)),
            pl.BlockSpec((NBLK_N, HF), lambda i: (i, 0)),
            pl.BlockSpec((NBLK_N, NF), lambda i: (i, 0)),
            pl.BlockSpec((NBLK_N, 1), lambda i: (i, 0)),
            full(HF, HID),
            full(1, HID),
            full(1, HID),
            full(2 * HID + 1, HID),
            full(1, HID),
        ],
        out_specs=[
            pl.BlockSpec((NBLK_N, POS_DIM), lambda i: (i, 0)),
            pl.BlockSpec((NBLK_N, HID), lambda i: (i, 0)),
            pl.BlockSpec((NBLK_N, HID), lambda i: (i, 0)),
            pl.BlockSpec((NBLK_N, HID), lambda i: (i, 0)),
            pl.BlockSpec((NBLK_N, POS_DIM), lambda i: (i, 0)),
        ],
        out_shape=[
            jax.ShapeDtypeStruct((N, POS_DIM), _f32),
            jax.ShapeDtypeStruct((N, HID), _f32),
            jax.ShapeDtypeStruct((N, HID), _f32),
            jax.ShapeDtypeStruct((N, HID), _f32),
            jax.ShapeDtypeStruct((N, POS_DIM), _f32),
        ],
    )(stats, t, cond, pos, h, eps, mask_c, We, Wt, Wcnd, Wm1, bm1)


def _sc_gather(tsrc, tdst, x0, x1, x2, src, dst):
    mesh = plsc.VectorSubcoreMesh(core_axis_name="c", subcore_axis_name="s")

    @functools.partial(
        pl.kernel,
        out_type=[jax.ShapeDtypeStruct((E, HID), _f32),
                  jax.ShapeDtypeStruct((E, HID), _f32),
                  jax.ShapeDtypeStruct((E,), _f32),
                  jax.ShapeDtypeStruct((E,), _f32),
                  jax.ShapeDtypeStruct((E,), _f32),
                  jax.ShapeDtypeStruct((E,), _f32)],
        mesh=mesh,
        scratch_types=[
            pltpu.VMEM((KE,), jnp.int32),
            pltpu.VMEM((KE,), jnp.int32),
            pltpu.VMEM((KE, HID), _f32),
            pltpu.VMEM((KE, HID), _f32),
            pltpu.VMEM((N,), _f32),
            pltpu.VMEM((N,), _f32),
            pltpu.VMEM((N,), _f32),
            pltpu.VMEM((KE,), _f32),
            pltpu.VMEM((KE,), _f32),
            pltpu.VMEM((KE,), _f32),
            pltpu.VMEM((KE,), _f32),
            pltpu.SemaphoreType.DMA,
            pltpu.SemaphoreType.DMA,
        ],
        compiler_params=pltpu.CompilerParams(use_tc_tiling_on_sc=True,
                                             needs_layout_passes=False),
    )
    def gather_k(tsrc_hbm, tdst_hbm, x0_hbm, x1_hbm, x2_hbm, src_hbm, dst_hbm,
                 gs_hbm, gd_hbm, rx_hbm, ry_hbm, rz_hbm, d2_hbm,
                 idx_s, idx_d, rows_s, rows_d, xv0, xv1, xv2,
                 rbx, rby, rbz, rb2, sem_s, sem_d):
        wid = lax.axis_index("s") * NC + lax.axis_index("c")
        base0 = wid * EPW
        pltpu.sync_copy(x0_hbm, xv0)
        pltpu.sync_copy(x1_hbm, xv1)
        pltpu.sync_copy(x2_hbm, xv2)

        def body(j, _):
            base = base0 + j * KE
            pltpu.sync_copy(src_hbm.at[pl.ds(base, KE)], idx_s)
            pltpu.sync_copy(dst_hbm.at[pl.ds(base, KE)], idx_d)
            cs = pltpu.async_copy(tsrc_hbm.at[idx_s], rows_s, sem_s)
            cd = pltpu.async_copy(tdst_hbm.at[idx_d], rows_d, sem_d)
            for g in range(KE // 16):
                sl = pl.ds(g * 16, 16)
                iv_s = idx_s[sl]
                iv_d = idx_d[sl]
                rx = (plsc.load_gather(xv0, [iv_s])
                      - plsc.load_gather(xv0, [iv_d]))
                ry = (plsc.load_gather(xv1, [iv_s])
                      - plsc.load_gather(xv1, [iv_d]))
                rz = (plsc.load_gather(xv2, [iv_s])
                      - plsc.load_gather(xv2, [iv_d]))
                rbx[sl] = rx
                rby[sl] = ry
                rbz[sl] = rz
                rb2[sl] = rx * rx + ry * ry + rz * rz
            cs.wait()
            cd.wait()
            pltpu.sync_copy(rows_s, gs_hbm.at[pl.ds(base, KE)])
            pltpu.sync_copy(rows_d, gd_hbm.at[pl.ds(base, KE)])
            pltpu.sync_copy(rbx, rx_hbm.at[pl.ds(base, KE)])
            pltpu.sync_copy(rby, ry_hbm.at[pl.ds(base, KE)])
            pltpu.sync_copy(rbz, rz_hbm.at[pl.ds(base, KE)])
            pltpu.sync_copy(rb2, d2_hbm.at[pl.ds(base, KE)])
            return 0

        lax.fori_loop(0, NBLK, body, 0)

    return gather_k(tsrc, tdst, x0, x1, x2, src, dst)


def _edge_body(gs_ref, gd_ref, rx_ref, ry_ref, rz_ref, d2_ref,
               Wm2_ref, bm2_ref, wx_ref, wd2_ref,
               m_ref, cx_ref, cy_ref, cz_ref):
    d2r = d2_ref[0]
    outer = lax.dot_general(d2r, wd2_ref[...],
                            (((0,), (0,)), ((), ())),
                            preferred_element_type=_f32)
    q = jnp.maximum(gs_ref[...] + gd_ref[...] + outer, 0.0)
    m = jnp.maximum(jnp.dot(q, Wm2_ref[...], preferred_element_type=_f32)
                    + bm2_ref[...], 0.0)
    srow = jnp.tanh(lax.dot_general(wx_ref[...], m,
                                    (((1,), (1,)), ((), ())),
                                    preferred_element_type=_f32))
    m_ref[...] = m
    cx_ref[0] = rx_ref[0] * srow
    cy_ref[0] = ry_ref[0] * srow
    cz_ref[0] = rz_ref[0] * srow


def _edge_mlp(gs, gd, rx, ry, rz, d2, Wm2, bm2, wx, wd2):
    row = pl.BlockSpec((1, 1, EBLK), lambda i: (i, 0, 0))
    return pl.pallas_call(
        _edge_body,
        grid=(NEB,),
        in_specs=[
            pl.BlockSpec((EBLK, HID), lambda i: (i, 0)),
            pl.BlockSpec((EBLK, HID), lambda i: (i, 0)),
            row, row, row, row,
            pl.BlockSpec((HID, HID), lambda i: (0, 0)),
            pl.BlockSpec((1, HID), lambda i: (0, 0)),
            pl.BlockSpec((1, HID), lambda i: (0, 0)),
            pl.BlockSpec((1, HID), lambda i: (0, 0)),
        ],
        out_specs=[pl.BlockSpec((EBLK, HID), lambda i: (i, 0)),
                   row, row, row],
        out_shape=[jax.ShapeDtypeStruct((E, HID), _f32),
                   jax.ShapeDtypeStruct((NEB, 1, EBLK), _f32),
                   jax.ShapeDtypeStruct((NEB, 1, EBLK), _f32),
                   jax.ShapeDtypeStruct((NEB, 1, EBLK), _f32)],
    )(gs, gd, rx, ry, rz, d2, Wm2, bm2, wx, wd2)


def _sc_scatter(rows, dst, zeros_n):
    mesh = plsc.VectorSubcoreMesh(core_axis_name="c", subcore_axis_name="s")

    @functools.partial(
        pl.kernel,
        out_type=jax.ShapeDtypeStruct((NC * N, HID), _f32),
        mesh=mesh,
        scratch_types=[
            pltpu.VMEM((KE,), jnp.int32),
            pltpu.VMEM((KE, HID), _f32),
            pltpu.MemorySpace.VMEM_SHARED((N, HID), _f32),
        ],
        compiler_params=pltpu.CompilerParams(use_tc_tiling_on_sc=True),
    )
    def scatter_k(rows_hbm, dst_hbm, zeros_hbm, out_hbm, idx_d, buf, acc):
        cid = lax.axis_index("c")
        sid = lax.axis_index("s")
        wid = sid * NC + cid
        base0 = wid * EPW

        @pl.when(sid == 0)
        def _():
            pltpu.sync_copy(zeros_hbm, acc)

        plsc.subcore_barrier()

        def body(j, _):
            base = base0 + j * KE
            pltpu.sync_copy(dst_hbm.at[pl.ds(base, KE)], idx_d)
            pltpu.sync_copy(rows_hbm.at[pl.ds(base, KE)], buf)
            pltpu.sync_copy(buf, acc.at[idx_d], add=True)
            return 0

        lax.fori_loop(0, NBLK, body, 0)
        plsc.subcore_barrier()

        @pl.when(sid == 0)
        def _():
            pltpu.sync_copy(acc, out_hbm.at[pl.ds(cid * N, N)])

    return scatter_k(rows, dst, zeros_n)


def _sc_cscatter(cx, cy, cz, dst):
    mesh = plsc.VectorSubcoreMesh(core_axis_name="c", subcore_axis_name="s")

    @functools.partial(
        pl.kernel,
        out_type=jax.ShapeDtypeStruct((NW * CW, N), _f32),
        mesh=mesh,
        scratch_types=[
            pltpu.VMEM((KE,), jnp.int32),
            pltpu.VMEM((KE,), _f32),
            pltpu.VMEM((KE,), _f32),
            pltpu.VMEM((KE,), _f32),
            pltpu.VMEM((CW, N), _f32),
        ],
        compiler_params=pltpu.CompilerParams(use_tc_tiling_on_sc=False,
                                             needs_layout_passes=False),
    )
    def cscatter_k(cx_hbm, cy_hbm, cz_hbm, dst_hbm, out_hbm,
                   idx_d, cbx, cby, cbz, acc):
        cid = lax.axis_index("c")
        sid = lax.axis_index("s")
        wid = sid * NC + cid
        base0 = wid * EPW

        zv = jnp.zeros((16,), _f32)
        cvecs = [jnp.full((16,), c, jnp.int32) for c in range(POS_DIM)]

        def zbody(j, _):
            for c in range(POS_DIM):
                acc[c, pl.ds(j * 16, 16)] = zv
            return 0

        lax.fori_loop(0, N // 16, zbody, 0)

        def body(j, _):
            base = base0 + j * KE
            pltpu.sync_copy(dst_hbm.at[pl.ds(base, KE)], idx_d)
            pltpu.sync_copy(cx_hbm.at[pl.ds(base, KE)], cbx)
            pltpu.sync_copy(cy_hbm.at[pl.ds(base, KE)], cby)
            pltpu.sync_copy(cz_hbm.at[pl.ds(base, KE)], cbz)
            for g in range(KE // 16):
                sl = pl.ds(g * 16, 16)
                iv_d = idx_d[sl]
                plsc.addupdate_scatter(acc, [cvecs[0], iv_d], cbx[sl])
                plsc.addupdate_scatter(acc, [cvecs[1], iv_d], cby[sl])
                plsc.addupdate_scatter(acc, [cvecs[2], iv_d], cbz[sl])
            return 0

        lax.fori_loop(0, NBLK, body, 0)
        pltpu.sync_copy(acc, out_hbm.at[pl.ds(wid * CW, CW)])

    return cscatter_k(cx, cy, cz, dst)


def _xstats_body(px_ref, mask_rw_ref, stats_ref, xagg_ref):
    xrow = jnp.sum(px_ref[...], axis=0)[:POS_DIM]
    msk_r = mask_rw_ref[...]
    onehotT = (lax.broadcasted_iota(jnp.int32, (B, N), 0)
               == msk_r).astype(_f32)
    xsum = lax.dot_general(onehotT, xrow, (((1,), (1,)), ((), ())),
                           preferred_element_type=_f32)
    cnt = jnp.sum(onehotT, axis=1, keepdims=True)
    stats_ref[...] = jnp.concatenate(
        [xsum, cnt, jnp.zeros((B, 4), _f32)], axis=1)
    eye3 = (lax.broadcasted_iota(jnp.int32, (POS_DIM, POS_DIM), 0)
            == lax.broadcasted_iota(
                jnp.int32, (POS_DIM, POS_DIM), 1)).astype(_f32)
    xagg_ref[...] = lax.dot_general(xrow, eye3, (((0,), (0,)), ((), ())),
                                    preferred_element_type=_f32)


def _xstats(px32, mask_rw):
    return pl.pallas_call(
        _xstats_body,
        grid=(1,),
        in_specs=[
            pl.BlockSpec((NW, CW, N), lambda i: (0, 0, 0)),
            pl.BlockSpec((1, N), lambda i: (0, 0)),
        ],
        out_specs=[pl.BlockSpec((B, 8), lambda i: (0, 0)),
                   pl.BlockSpec((N, POS_DIM), lambda i: (0, 0))],
        out_shape=[jax.ShapeDtypeStruct((B, 8), _f32),
                   jax.ShapeDtypeStruct((N, POS_DIM), _f32)],
    )(px32, mask_rw)


def _post_body(stats_ref, pa_ref, pb_ref, xagg_ref, h0_ref, Wn1_ref,
               bn1_ref, Wn2_ref, bn2_ref, Wo_ref, bo_ref, mask_c_ref,
               mask_r_ref, epsc_ref, epsh_ref, err_ref, epos_ref, eh_ref):
    i = pl.program_id(0)

    @pl.when(i == 0)
    def _():
        err_ref[...] = jnp.zeros_like(err_ref)

    agg = pa_ref[...] + pb_ref[...]
    xagg = xagg_ref[...]

    h0 = h0_ref[...]
    Wn1 = Wn1_ref[...]
    pre = (jnp.dot(h0, Wn1[0:HID], preferred_element_type=_f32)
           + jnp.dot(agg, Wn1[HID:2 * HID], preferred_element_type=_f32)
           + bn1_ref[...])
    h1 = h0 + jnp.dot(jnp.maximum(pre, 0.0), Wn2_ref[...],
                      preferred_element_type=_f32) + bn2_ref[...]

    stats = stats_ref[...]
    cnt = jnp.maximum(stats[:, 3:4], 1.0)
    xm = stats[:, 0:3] / cnt

    msk_c = mask_c_ref[...]
    onehot = (msk_c == lax.broadcasted_iota(
        jnp.int32, (NBLK_N, B), 1)).astype(_f32)
    msk_r = mask_r_ref[0]
    onehotT = (lax.broadcasted_iota(jnp.int32, (B, NBLK_N), 0)
               == msk_r).astype(_f32)

    eps_pos = xagg - jnp.dot(onehot, xm, preferred_element_type=_f32)
    eps_h = jnp.dot(h1, Wo_ref[...], preferred_element_type=_f32) + bo_ref[...]

    perr = epsc_ref[...] - eps_pos
    herr = epsh_ref[...] - eps_h
    err = (jnp.sum(perr * perr, axis=1, keepdims=True)
           + jnp.sum(herr * herr, axis=1, keepdims=True))
    err_ref[...] += jnp.dot(onehotT, err, preferred_element_type=_f32)
    epos_ref[...] = eps_pos
    eh_ref[...] = eps_h


def _post(stats, pm, xagg, h0, Wn1, bn1, Wn2, bn2, Wo, bo, mask_c, mask_r,
          epsc, epsh):
    full = lambda r, c: pl.BlockSpec((r, c), lambda i: (0, 0))
    return pl.pallas_call(
        _post_body,
        grid=(NGB,),
        in_specs=[
            full(B, 8),
            pl.BlockSpec((NBLK_N, HID), lambda i: (i, 0)),
            pl.BlockSpec((NBLK_N, HID), lambda i: (i + NGB, 0)),
            pl.BlockSpec((NBLK_N, POS_DIM), lambda i: (i, 0)),
            pl.BlockSpec((NBLK_N, HID), lambda i: (i, 0)),
            full(2 * HID, HID),
            full(1, HID),
            full(HID, HID),
            full(1, HID),
            full(HID, HF),
            full(1, HF),
            pl.BlockSpec((NBLK_N, 1), lambda i: (i, 0)),
            pl.BlockSpec((1, 1, NBLK_N), lambda i: (i, 0, 0)),
            pl.BlockSpec((NBLK_N, POS_DIM), lambda i: (i, 0)),
            pl.BlockSpec((NBLK_N, HF), lambda i: (i, 0)),
        ],
        out_specs=[
            pl.BlockSpec((B, 1), lambda i: (0, 0)),
            pl.BlockSpec((NBLK_N, POS_DIM), lambda i: (i, 0)),
            pl.BlockSpec((NBLK_N, HF), lambda i: (i, 0)),
        ],
        out_shape=[
            jax.ShapeDtypeStruct((B, 1), _f32),
            jax.ShapeDtypeStruct((N, POS_DIM), _f32),
            jax.ShapeDtypeStruct((N, HF), _f32),
        ],
    )(stats, pm, pm, xagg, h0, Wn1, bn1, Wn2, bn2, Wo, bo, mask_c, mask_r,
      epsc, epsh)


def kernel(pos, h, conditions, t, eps, We, Wt, Wcnd, Wm1, bm1, Wm2, bm2,
           Wn1, bn1, Wn2, bn2, Wx, Wo, bo, mask, edge_index):
    mask = mask.astype(jnp.int32)
    mask_c = mask.reshape(N, 1)
    mask_r = mask.reshape(NGB, 1, NBLK_N)
    src = edge_index[0].astype(jnp.int32)
    dst = edge_index[1].astype(jnp.int32)

    stats, snr = _stats(pos, eps[:, :POS_DIM], mask_r, t)
    epsc, h0, tsrc, tdst, zx = _pre(
        stats, t, conditions, pos, h, eps, mask_c,
        We, Wt.reshape(1, HID), Wcnd.reshape(1, HID), Wm1, bm1.reshape(1, HID))

    gs, gd, relx, rely, relz, d2 = _sc_gather(
        tsrc, tdst, zx[:, 0], zx[:, 1], zx[:, 2], src, dst)
    mrows, cx, cy, cz = _edge_mlp(
        gs, gd,
        relx.reshape(NEB, 1, EBLK), rely.reshape(NEB, 1, EBLK),
        relz.reshape(NEB, 1, EBLK), d2.reshape(NEB, 1, EBLK),
        Wm2, bm2.reshape(1, HID), Wx.reshape(1, HID),
        Wm1[2 * HID].reshape(1, HID))
    pm = _sc_scatter(mrows, dst, jnp.zeros((N, HID), _f32))
    px32 = _sc_cscatter(cx.reshape(E), cy.reshape(E), cz.reshape(E), dst)

    xstats, xagg = _xstats(px32.reshape(NW, CW, N), mask.reshape(1, N))
    error_t, eps_pos, eps_h = _post(
        xstats, pm, xagg, h0, Wn1, bn1.reshape(1, HID), Wn2,
        bn2.reshape(1, HID),
        Wo, bo.reshape(1, HF), mask_c, mask_r, epsc, eps[:, POS_DIM:])

    net_eps_xh = jnp.concatenate([eps_pos, eps_h], axis=1)
    eps_xh = jnp.concatenate([epsc, eps[:, POS_DIM:]], axis=1)
    return (error_t.reshape(B), snr.reshape(B), net_eps_xh, eps_xh)
